# baseline (speedup 1.0000x reference)
import functools

import jax
import jax.numpy as jnp
from jax import lax
from jax.experimental import pallas as pl
from jax.experimental.pallas import tpu as pltpu

B = 16384
L = 200
RBLK = 1024
GRID = B // RBLK


def _tc_kernel(node_ref, items_ref, out_ref):
    node = node_ref[...]
    items = items_ref[...]
    col = lax.broadcasted_iota(jnp.int32, (RBLK, L), 1)
    c = jnp.clip(node - 1, 0, L - 1)
    pick = (col == c) & (node != 0)
    sel = jnp.where(pick, items, jnp.int32(-1))
    out_ref[...] = jnp.max(sel, axis=1)


@jax.jit
def _decode(node, items):
    return pl.pallas_call(
        _tc_kernel,
        grid=(GRID,),
        in_specs=[
            pl.BlockSpec((RBLK, 1), lambda i: (i, 0)),
            pl.BlockSpec((RBLK, L), lambda i: (i, 0)),
        ],
        out_specs=pl.BlockSpec((RBLK,), lambda i: (i,)),
        out_shape=jax.ShapeDtypeStruct((B,), jnp.int32),
        compiler_params=pltpu.CompilerParams(
            dimension_semantics=("arbitrary",),
        ),
    )(node, items)


def kernel(x_dummy, current_node, item_ids):
    node = current_node.astype(jnp.int32)
    return _decode(node, item_ids.astype(jnp.int32)).astype(item_ids.dtype)
